# baseline (speedup 1.0000x reference)
import functools

import jax
import jax.numpy as jnp
from jax import lax
from jax.experimental import pallas as pl
from jax.experimental.pallas import tpu as pltpu
from jax.experimental.pallas import tpu_sc as plsc

D = 64
K = 1024
B = 16
ROWS = B * 1024
NHALF = 2
HB = B // NHALF
HROWS = ROWS // NHALF
BLK = 1024
SUB = 128
NSUB = BLK // SUB
GRP = 128
NGRP = K // GRP
BETA = 0.25


def _tc_body(z_ref, se_ref, emb_m2t_ref, idx_ref, loss_ref):
    i = pl.program_id(0)
    z = z_ref[0]
    sz = jnp.sum(z * z, axis=1, keepdims=True)

    lane = lax.broadcasted_iota(jnp.int32, (SUB, GRP), 1)
    loss_part = None
    for r in range(NSUB):
        zr = z[r * SUB:(r + 1) * SUB]
        szr = sz[r * SUB:(r + 1) * SUB]
        val = None
        idx = None
        for c in range(NGRP):
            mm2 = jnp.dot(zr, emb_m2t_ref[:, c * GRP:(c + 1) * GRP],
                          preferred_element_type=jnp.float32)
            d_c = (szr + se_ref[:, c * GRP:(c + 1) * GRP]) + mm2
            if c == 0:
                val, idx = d_c, lane
            else:
                take = d_c < val
                val = jnp.where(take, d_c, val)
                idx = jnp.where(take, lane + c * GRP, idx)
        m = jnp.min(val, axis=1, keepdims=True)
        jmin = jnp.min(jnp.where(val == m, idx, K), axis=1)
        idx_ref[0, 0, r * SUB:(r + 1) * SUB] = jmin
        part = jnp.sum(m)
        loss_part = part if loss_part is None else loss_part + part

    @pl.when(i == 0)
    def _init():
        loss_ref[0, 0] = 0.0

    loss_ref[0, 0] += loss_part


def _make_tc_call(half):
    return pl.pallas_call(
        _tc_body,
        grid=(HB,),
        in_specs=[
            pl.BlockSpec((1, BLK, D), lambda i: (i + half * HB, 0, 0)),
            pl.BlockSpec((1, K), lambda i: (0, 0)),
            pl.BlockSpec((D, K), lambda i: (0, 0)),
        ],
        out_specs=[
            pl.BlockSpec((1, 1, BLK), lambda i: (i, 0, 0)),
            pl.BlockSpec(memory_space=pltpu.SMEM),
        ],
        out_shape=[
            jax.ShapeDtypeStruct((HB, 1, BLK), jnp.int32),
            jax.ShapeDtypeStruct((1, 1), jnp.float32),
        ],
    )


_tc_calls = [_make_tc_call(h) for h in range(NHALF)]


_NC, _NS = 2, 16
_NW = _NC * _NS
BPW = HROWS // _NW
CHUNK = 128
NCH = BPW // CHUNK


@functools.cache
def _sc_gather_call():
    mesh = plsc.VectorSubcoreMesh(
        core_axis_name="c", subcore_axis_name="s")

    @functools.partial(
        pl.kernel,
        out_type=jax.ShapeDtypeStruct((HB, 1024, D), jnp.float32),
        mesh=mesh,
        scratch_types=[
            [pltpu.VMEM((CHUNK,), jnp.int32) for _ in range(NCH)],
            [pltpu.VMEM((CHUNK, D), jnp.float32) for _ in range(NCH)],
            pltpu.SemaphoreType.DMA,
        ],
        compiler_params=pltpu.CompilerParams(use_tc_tiling_on_sc=False),
    )
    def _sc_gather(emb_hbm, idx_hbm, out_hbm, idx_bufs, row_bufs, sem):
        wid = lax.axis_index("s") * _NC + lax.axis_index("c")
        base = wid * BPW
        for j in range(NCH):
            pltpu.sync_copy(
                idx_hbm.at[pl.ds(base + j * CHUNK, CHUNK)], idx_bufs[j])
        copies = [
            pltpu.async_copy(emb_hbm.at[idx_bufs[j]], row_bufs[j], sem)
            for j in range(NCH)
        ]
        for j in range(NCH):
            copies[j].wait()
            g = base + j * CHUNK
            pltpu.sync_copy(
                row_bufs[j], out_hbm.at[g // 1024, pl.ds(g % 1024, CHUNK)])

    return _sc_gather


def kernel(z, emb_w):
    se = jnp.sum(emb_w ** 2, axis=1).reshape(1, K)
    emb_m2t = emb_w.T * -2.0
    sc = _sc_gather_call()
    zq_halves = []
    loss_sum = None
    for h in range(NHALF):
        idx3d, loss2d = _tc_calls[h](z, se, emb_m2t)
        zq_halves.append(sc(emb_w, idx3d.reshape(HROWS)))
        part = loss2d[0, 0]
        loss_sum = part if loss_sum is None else loss_sum + part
    zq = jnp.concatenate(zq_halves, axis=0)
    loss = loss_sum * ((1.0 + BETA) / (ROWS * D))
    return zq, loss


# speedup vs baseline: 1.1235x; 1.1235x over previous
import functools

import jax
import jax.numpy as jnp
from jax import lax
from jax.experimental import pallas as pl
from jax.experimental.pallas import tpu as pltpu
from jax.experimental.pallas import tpu_sc as plsc

D = 64
K = 1024
B = 16
ROWS = B * 1024
NHALF = 2
HB = B // NHALF
HROWS = ROWS // NHALF
BLK = 1024
SUB = 128
NSUB = BLK // SUB
GRP = 128
NGRP = K // GRP
BETA = 0.25


def _tc_body(z_ref, se_ref, emb_m2t_ref, idx_ref, loss_ref):
    i = pl.program_id(0)
    z = z_ref[0]
    sz = jnp.sum(z * z, axis=1, keepdims=True)

    lane = lax.broadcasted_iota(jnp.int32, (SUB, GRP), 1)
    loss_part = None
    for r in range(NSUB):
        zr = z[r * SUB:(r + 1) * SUB]
        szr = sz[r * SUB:(r + 1) * SUB]
        val = None
        idx = None
        for c in range(NGRP):
            mm2 = jnp.dot(zr, emb_m2t_ref[:, c * GRP:(c + 1) * GRP],
                          preferred_element_type=jnp.float32)
            d_c = (szr + se_ref[:, c * GRP:(c + 1) * GRP]) + mm2
            if c == 0:
                val, idx = d_c, lane
            else:
                take = d_c < val
                val = jnp.where(take, d_c, val)
                idx = jnp.where(take, lane + c * GRP, idx)
        m = jnp.min(val, axis=1, keepdims=True)
        jmin = jnp.min(jnp.where(val == m, idx, K), axis=1)
        idx_ref[0, 0, r * SUB:(r + 1) * SUB] = jmin
        part = jnp.sum(m)
        loss_part = part if loss_part is None else loss_part + part

    @pl.when(i == 0)
    def _init():
        loss_ref[0, 0] = 0.0

    loss_ref[0, 0] += loss_part


def _make_tc_call(half):
    return pl.pallas_call(
        _tc_body,
        grid=(HB,),
        in_specs=[
            pl.BlockSpec((1, BLK, D), lambda i: (i + half * HB, 0, 0)),
            pl.BlockSpec((1, K), lambda i: (0, 0)),
            pl.BlockSpec((D, K), lambda i: (0, 0)),
        ],
        out_specs=[
            pl.BlockSpec((1, 1, BLK), lambda i: (i, 0, 0)),
            pl.BlockSpec(memory_space=pltpu.SMEM),
        ],
        out_shape=[
            jax.ShapeDtypeStruct((HB, 1, BLK), jnp.int32),
            jax.ShapeDtypeStruct((1, 1), jnp.float32),
        ],
    )


_tc_calls = [_make_tc_call(h) for h in range(NHALF)]


_NC, _NS = 2, 16
_NW = _NC * _NS
SPB = _NW // HB
DPW = D // SPB
L = 16
RV = 1024 // L


@functools.cache
def _sc_gather_call():
    mesh = plsc.VectorSubcoreMesh(
        core_axis_name="c", subcore_axis_name="s")

    @functools.partial(
        pl.kernel,
        out_type=jax.ShapeDtypeStruct((HB * D, 1024), jnp.float32),
        mesh=mesh,
        scratch_types=[
            pltpu.VMEM((1024,), jnp.int32),
            pltpu.VMEM((DPW * 1024,), jnp.float32),
            pltpu.VMEM((DPW, 1024), jnp.float32),
        ],
        compiler_params=pltpu.CompilerParams(
            use_tc_tiling_on_sc=False, needs_layout_passes=False),
    )
    def _sc_gather(embt_hbm, idx_hbm, out_hbm, idx_t, embt_t, out_t):
        wid = lax.axis_index("s") * _NC + lax.axis_index("c")
        b = wid // SPB
        p = wid % SPB
        pltpu.sync_copy(idx_hbm.at[pl.ds(b * 1024, 1024)], idx_t)
        pltpu.sync_copy(embt_hbm.at[pl.ds(p * DPW * 1024, DPW * 1024)], embt_t)

        def body(v, carry):
            iv = idx_t[pl.ds(v * L, L)]
            for d in range(DPW):
                out_t[d, pl.ds(v * L, L)] = plsc.load_gather(
                    embt_t, [iv + d * 1024])
            return carry

        lax.fori_loop(0, RV, body, 0)
        pltpu.sync_copy(out_t, out_hbm.at[pl.ds(b * D + p * DPW, DPW)])

    return _sc_gather


def kernel(z, emb_w):
    se = jnp.sum(emb_w ** 2, axis=1).reshape(1, K)
    emb_m2t = emb_w.T * -2.0
    sc = _sc_gather_call()
    zq_halves = []
    loss_sum = None
    embt = emb_w.T.reshape(D * K)
    for h in range(NHALF):
        idx3d, loss2d = _tc_calls[h](z, se, emb_m2t)
        zq_halves.append(sc(embt, idx3d.reshape(HROWS)))
        part = loss2d[0, 0]
        loss_sum = part if loss_sum is None else loss_sum + part
    zq_t = jnp.concatenate(zq_halves, axis=0).reshape(B, D, 1024)
    zq = jnp.swapaxes(zq_t, 1, 2)
    loss = loss_sum * ((1.0 + BETA) / (ROWS * D))
    return zq, loss


# speedup vs baseline: 1.1680x; 1.0396x over previous
import functools

import jax
import jax.numpy as jnp
from jax import lax
from jax.experimental import pallas as pl
from jax.experimental.pallas import tpu as pltpu
from jax.experimental.pallas import tpu_sc as plsc

D = 64
K = 1024
B = 16
ROWS = B * 1024
NHALF = 2
HB = B // NHALF
HROWS = ROWS // NHALF
BLK = 1024
SUB = 128
NSUB = BLK // SUB
GRP = 128
NGRP = K // GRP
BETA = 0.25


def _tc_body(half, z_hbm, emb_m2t_ref, idx_ref, loss_ref, z_buf, sem):
    i = pl.program_id(0)

    def z_copy(step, slot):
        return pltpu.make_async_copy(
            z_hbm.at[pl.ds(half * HB + step, 1)], z_buf.at[slot],
            sem.at[slot])

    @pl.when(i == 0)
    def _prologue():
        z_copy(0, 0).start()

    @pl.when(i + 1 < HB)
    def _prefetch():
        z_copy(i + 1, (i + 1) % 2).start()

    z_copy(i, i % 2).wait()
    z = z_buf[i % 2, 0]
    sz = jnp.sum(z * z, axis=1, keepdims=True)
    m2 = emb_m2t_ref[...]
    se = jnp.sum(m2 * m2, axis=0, keepdims=True) * 0.25

    lane = lax.broadcasted_iota(jnp.int32, (SUB, GRP), 1)
    loss_part = None
    for r in range(NSUB):
        zr = z[r * SUB:(r + 1) * SUB]
        szr = sz[r * SUB:(r + 1) * SUB]
        val = None
        idx = None
        for c in range(NGRP):
            mm2 = jnp.dot(zr, m2[:, c * GRP:(c + 1) * GRP],
                          preferred_element_type=jnp.float32)
            d_c = (szr + se[:, c * GRP:(c + 1) * GRP]) + mm2
            if c == 0:
                val, idx = d_c, lane
            else:
                take = d_c < val
                val = jnp.where(take, d_c, val)
                idx = jnp.where(take, lane + c * GRP, idx)
        m = jnp.min(val, axis=1, keepdims=True)
        jmin = jnp.min(jnp.where(val == m, idx, K), axis=1)
        idx_ref[0, 0, r * SUB:(r + 1) * SUB] = jmin
        part = jnp.sum(m)
        loss_part = part if loss_part is None else loss_part + part

    @pl.when(i == 0)
    def _init():
        loss_ref[0, 0] = 0.0

    loss_ref[0, 0] += loss_part


def _make_tc_call(half):
    return pl.pallas_call(
        functools.partial(_tc_body, half),
        grid=(HB,),
        in_specs=[
            pl.BlockSpec(memory_space=pltpu.HBM),
            pl.BlockSpec((D, K), lambda i: (0, 0)),
        ],
        out_specs=[
            pl.BlockSpec((1, 1, BLK), lambda i: (i, 0, 0)),
            pl.BlockSpec(memory_space=pltpu.SMEM),
        ],
        out_shape=[
            jax.ShapeDtypeStruct((HB, 1, BLK), jnp.int32),
            jax.ShapeDtypeStruct((1, 1), jnp.float32),
        ],
        scratch_shapes=[
            pltpu.VMEM((2, 1, BLK, D), jnp.float32),
            pltpu.SemaphoreType.DMA((2,)),
        ],
    )


_tc_calls = [_make_tc_call(h) for h in range(NHALF)]


_NC, _NS = 2, 16
_NW = _NC * _NS
SPB = _NW // HB
DPW = D // SPB
L = 16
RV = 1024 // L


@functools.cache
def _sc_gather_call():
    mesh = plsc.VectorSubcoreMesh(
        core_axis_name="c", subcore_axis_name="s")

    @functools.partial(
        pl.kernel,
        out_type=jax.ShapeDtypeStruct((HB * D, 1024), jnp.float32),
        mesh=mesh,
        scratch_types=[
            pltpu.VMEM((1024,), jnp.int32),
            pltpu.VMEM((DPW * 1024,), jnp.float32),
            pltpu.VMEM((DPW, 1024), jnp.float32),
            pltpu.SemaphoreType.DMA,
            pltpu.SemaphoreType.DMA,
        ],
        compiler_params=pltpu.CompilerParams(
            use_tc_tiling_on_sc=False, needs_layout_passes=False),
    )
    def _sc_gather(embt_hbm, idx_hbm, out_hbm, idx_t, embt_t, out_t,
                   sem1, sem2):
        wid = lax.axis_index("s") * _NC + lax.axis_index("c")
        b = wid // SPB
        p = wid % SPB
        c1 = pltpu.async_copy(
            idx_hbm.at[pl.ds(b * 1024, 1024)], idx_t, sem1)
        c2 = pltpu.async_copy(
            embt_hbm.at[pl.ds(p * DPW * 1024, DPW * 1024)], embt_t, sem2)
        c1.wait()
        c2.wait()

        def body(v, carry):
            iv = idx_t[pl.ds(v * L, L)]
            for d in range(DPW):
                out_t[d, pl.ds(v * L, L)] = plsc.load_gather(
                    embt_t, [iv + d * 1024])
            return carry

        lax.fori_loop(0, RV, body, 0)
        pltpu.sync_copy(out_t, out_hbm.at[pl.ds(b * D + p * DPW, DPW)])

    return _sc_gather


def kernel(z, emb_w):
    emb_m2t = emb_w.T * -2.0
    sc = _sc_gather_call()
    zq_halves = []
    loss_sum = None
    embt = emb_w.T.reshape(D * K)
    for h in range(NHALF):
        idx3d, loss2d = _tc_calls[h](z, emb_m2t)
        zq_halves.append(sc(embt, idx3d.reshape(HROWS)))
        part = loss2d[0, 0]
        loss_sum = part if loss_sum is None else loss_sum + part
    zq_t = jnp.concatenate(zq_halves, axis=0).reshape(B, D, 1024)
    zq = jnp.swapaxes(zq_t, 1, 2)
    loss = loss_sum * ((1.0 + BETA) / (ROWS * D))
    return zq, loss


# speedup vs baseline: 1.1774x; 1.0081x over previous
import functools

import jax
import jax.numpy as jnp
from jax import lax
from jax.experimental import pallas as pl
from jax.experimental.pallas import tpu as pltpu
from jax.experimental.pallas import tpu_sc as plsc

D = 64
K = 1024
B = 16
ROWS = B * 1024
NHALF = 2
HB = B // NHALF
HROWS = ROWS // NHALF
BLK = 1024
SUB = 128
NSUB = BLK // SUB
GRP = 128
NGRP = K // GRP
BETA = 0.25


def _tc_body(half, z_hbm, emb_m2t_ref, idx_ref, loss_ref, z_buf, sem):
    i = pl.program_id(0)

    def z_copy(step, slot):
        return pltpu.make_async_copy(
            z_hbm.at[pl.ds(half * HB + step, 1)], z_buf.at[slot],
            sem.at[slot])

    @pl.when(i == 0)
    def _prologue():
        z_copy(0, 0).start()

    @pl.when(i + 1 < HB)
    def _prefetch():
        z_copy(i + 1, (i + 1) % 2).start()

    z_copy(i, i % 2).wait()
    z = z_buf[i % 2, 0]
    sz = jnp.sum(z * z, axis=1, keepdims=True)
    m2 = emb_m2t_ref[...]
    se = jnp.sum(m2 * m2, axis=0, keepdims=True) * 0.25

    lane = lax.broadcasted_iota(jnp.int32, (SUB, GRP), 1)
    loss_part = None
    for r in range(NSUB):
        zr = z[r * SUB:(r + 1) * SUB]
        szr = sz[r * SUB:(r + 1) * SUB]
        val = None
        idx = None
        for c in range(NGRP):
            mm2 = jnp.dot(zr, m2[:, c * GRP:(c + 1) * GRP],
                          preferred_element_type=jnp.float32)
            d_c = (szr + se[:, c * GRP:(c + 1) * GRP]) + mm2
            if c == 0:
                val, idx = d_c, lane
            else:
                take = d_c < val
                val = jnp.where(take, d_c, val)
                idx = jnp.where(take, lane + c * GRP, idx)
        m = jnp.min(val, axis=1, keepdims=True)
        jmin = jnp.min(jnp.where(val == m, idx, K), axis=1)
        idx_ref[0, 0, r * SUB:(r + 1) * SUB] = jmin
        part = jnp.sum(m)
        loss_part = part if loss_part is None else loss_part + part

    @pl.when(i == 0)
    def _init():
        loss_ref[0, 0] = 0.0

    loss_ref[0, 0] += loss_part


def _make_tc_call(half):
    return pl.pallas_call(
        functools.partial(_tc_body, half),
        grid=(HB,),
        in_specs=[
            pl.BlockSpec(memory_space=pltpu.HBM),
            pl.BlockSpec((D, K), lambda i: (0, 0)),
        ],
        out_specs=[
            pl.BlockSpec((1, 1, BLK), lambda i: (i, 0, 0)),
            pl.BlockSpec(memory_space=pltpu.SMEM),
        ],
        out_shape=[
            jax.ShapeDtypeStruct((HB, 1, BLK), jnp.int32),
            jax.ShapeDtypeStruct((1, 1), jnp.float32),
        ],
        scratch_shapes=[
            pltpu.VMEM((2, 1, BLK, D), jnp.float32),
            pltpu.SemaphoreType.DMA((2,)),
        ],
    )


_tc_calls = [_make_tc_call(h) for h in range(NHALF)]


_NC, _NS = 2, 16
_NW = _NC * _NS
SPB = _NW // HB
DPW = D // SPB
L = 16
RV = 1024 // L


@functools.cache
def _sc_gather_call():
    mesh = plsc.VectorSubcoreMesh(
        core_axis_name="c", subcore_axis_name="s")

    @functools.partial(
        pl.kernel,
        out_type=jax.ShapeDtypeStruct((HB * D, 1024), jnp.float32),
        mesh=mesh,
        scratch_types=[
            pltpu.VMEM((1024,), jnp.int32),
            pltpu.VMEM((DPW * 1024,), jnp.float32),
            pltpu.VMEM((DPW, 1024), jnp.float32),
            pltpu.SemaphoreType.DMA,
            pltpu.SemaphoreType.DMA,
        ],
        compiler_params=pltpu.CompilerParams(
            use_tc_tiling_on_sc=False, needs_layout_passes=False),
    )
    def _sc_gather(embt_hbm, idx_hbm, out_hbm, idx_t, embt_t, out_t,
                   sem1, sem2):
        wid = lax.axis_index("s") * _NC + lax.axis_index("c")
        b = wid // SPB
        p = wid % SPB
        c1 = pltpu.async_copy(
            idx_hbm.at[pl.ds(b * 1024, 1024)], idx_t, sem1)
        c2 = pltpu.async_copy(
            embt_hbm.at[pl.ds(p * DPW * 1024, DPW * 1024)], embt_t, sem2)
        c1.wait()
        c2.wait()

        def body(v2, carry):
            for u in range(2):
                v = v2 * 2 + u
                iv = idx_t[pl.ds(v * L, L)]
                for d in range(DPW):
                    out_t[d, pl.ds(v * L, L)] = plsc.load_gather(
                        embt_t, [iv + d * 1024])
            return carry

        lax.fori_loop(0, RV // 2, body, 0)
        pltpu.sync_copy(out_t, out_hbm.at[pl.ds(b * D + p * DPW, DPW)])

    return _sc_gather


def kernel(z, emb_w):
    emb_m2t = emb_w.T * -2.0
    sc = _sc_gather_call()
    zq_halves = []
    loss_sum = None
    embt = emb_w.T.reshape(D * K)
    for h in range(NHALF):
        idx3d, loss2d = _tc_calls[h](z, emb_m2t)
        zq_halves.append(sc(embt, idx3d.reshape(HROWS)))
        part = loss2d[0, 0]
        loss_sum = part if loss_sum is None else loss_sum + part
    zq_t = jnp.concatenate(zq_halves, axis=0).reshape(B, D, 1024)
    zq = jnp.swapaxes(zq_t, 1, 2)
    loss = loss_sum * ((1.0 + BETA) / (ROWS * D))
    return zq, loss
